# baseline (speedup 1.0000x reference)
import functools

import jax
import jax.numpy as jnp
from jax import lax
from jax.experimental import pallas as pl
from jax.experimental.pallas import tpu as pltpu
from jax.experimental.pallas import tpu_sc as plsc

E = 16
D = 768
DFF = 2048
RANK = 16
SCALING = 32.0 / 16.0
GAMMA = 0.1
T = 2048
BT = 64
NB = T // BT + E - 1
TPAD = NB * BT
NW = 32
TW = T // NW


def _era(h):
    return jax.nn.gelu(h, approximate=True) + GAMMA * jax.nn.softplus(h)


def _nt(a, b):
    return jax.lax.dot_general(a, b, (((1,), (1,)), ((), ())),
                               preferred_element_type=jnp.float32)


def _ffn(h, gbw, gdw, guw, ubw, udw, uuw, dw):
    gate = _era(_nt(h, gbw) + SCALING * _nt(_nt(h, gdw), guw))
    up = _nt(h, ubw) + SCALING * _nt(_nt(h, udw), uuw)
    return _nt(gate * up, dw)


def _router_body(x_ref, rw_ref, dest_ref, be_ref, aux_ref):
    logits = _nt(x_ref[...], rw_ref[...])
    m = jnp.max(logits, axis=-1, keepdims=True)
    p = jnp.exp(logits - m)
    probs = p / jnp.sum(p, axis=-1, keepdims=True)
    pmax = jnp.max(probs, axis=-1, keepdims=True)
    lanes = lax.broadcasted_iota(jnp.int32, (T, E), 1)
    idx = jnp.min(jnp.where(probs >= pmax, lanes, E), axis=-1, keepdims=True)
    oneh = (lanes == idx).astype(jnp.float32)

    counts = jnp.sum(oneh, axis=0, keepdims=True)
    p_sum = jnp.sum(probs, axis=0, keepdims=True)
    aux_ref[0, 0] = E * jnp.sum((counts / T) * (p_sum / T))

    pc = jnp.ceil(counts * (1.0 / BT)) * BT
    ii = lax.broadcasted_iota(jnp.int32, (E, E), 0)
    jj = lax.broadcasted_iota(jnp.int32, (E, E), 1)
    upper = (ii < jj).astype(jnp.float32)
    po_excl = jnp.dot(pc, upper, preferred_element_type=jnp.float32)

    ci = lax.broadcasted_iota(jnp.int32, (BT, BT), 0)
    cj = lax.broadcasted_iota(jnp.int32, (BT, BT), 1)
    lstrict = (cj < ci).astype(jnp.float32)
    carry = jnp.zeros((1, E), jnp.float32)
    pieces = []
    for c in range(T // BT):
        oc = oneh[c * BT:(c + 1) * BT]
        cum = jnp.dot(lstrict, oc, preferred_element_type=jnp.float32) + carry
        pieces.append(jnp.sum(oc * (cum + po_excl), axis=-1, keepdims=True))
        carry = carry + jnp.sum(oc, axis=0, keepdims=True)
    dest = jnp.concatenate(pieces, axis=0)
    dest_ref[...] = jnp.broadcast_to(dest, (T, 128)).astype(jnp.int32)

    total_pad = jnp.sum(pc)
    bi = lax.broadcasted_iota(jnp.int32, (NB + 1, 1), 0)
    ge = (po_excl <= (bi * BT).astype(jnp.float32)).astype(jnp.float32)
    be = jnp.sum(ge, axis=-1, keepdims=True) - 1.0
    nb_live = total_pad * (1.0 / BT)
    be = jnp.where(bi == NB, nb_live, be)
    be_ref[...] = jnp.broadcast_to(be, (NB + 1, 128)).astype(jnp.int32)


def _router_plan(x2, rw):
    return pl.pallas_call(
        _router_body,
        in_specs=[pl.BlockSpec((T, D), lambda: (0, 0)),
                  pl.BlockSpec((E, D), lambda: (0, 0))],
        out_specs=[
            pl.BlockSpec((T, 128), lambda: (0, 0)),
            pl.BlockSpec((NB + 1, 128), lambda: (0, 0)),
            pl.BlockSpec((1, 1), lambda: (0, 0), memory_space=pltpu.SMEM),
        ],
        out_shape=[
            jax.ShapeDtypeStruct((T, 128), jnp.int32),
            jax.ShapeDtypeStruct((NB + 1, 128), jnp.int32),
            jax.ShapeDtypeStruct((1, 1), jnp.float32),
        ],
    )(x2, rw)


def _sc_scratch():
    return [pltpu.VMEM((TW,), jnp.int32),
            pltpu.VMEM((TW, D), jnp.float32),
            pltpu.SemaphoreType.DMA]


@functools.lru_cache(maxsize=None)
def _make_sc_movers():
    mesh = plsc.VectorSubcoreMesh(core_axis_name="c", subcore_axis_name="s")

    @functools.partial(pl.kernel, mesh=mesh,
                       out_type=jax.ShapeDtypeStruct((TPAD, D), jnp.float32),
                       scratch_types=_sc_scratch())
    def scatter_k(x_hbm, dest_hbm, xs_hbm, idx_v, rows_v, sem):
        wid = lax.axis_index("s") * 2 + lax.axis_index("c")
        base = wid * TW
        pltpu.sync_copy(dest_hbm.at[pl.ds(base, TW)], idx_v)
        pltpu.sync_copy(x_hbm.at[pl.ds(base, TW)], rows_v)
        pltpu.async_copy(rows_v, xs_hbm.at[idx_v], sem).wait()

    @functools.partial(pl.kernel, mesh=mesh,
                       out_type=jax.ShapeDtypeStruct((T, D), jnp.float32),
                       scratch_types=_sc_scratch())
    def gather_k(ys_hbm, dest_hbm, out_hbm, idx_v, rows_v, sem):
        wid = lax.axis_index("s") * 2 + lax.axis_index("c")
        base = wid * TW
        pltpu.sync_copy(dest_hbm.at[pl.ds(base, TW)], idx_v)
        pltpu.async_copy(ys_hbm.at[idx_v], rows_v, sem).wait()
        pltpu.sync_copy(rows_v, out_hbm.at[pl.ds(base, TW)])

    return scatter_k, gather_k


def _sc_scatter(x2, dest):
    return _make_sc_movers()[0](x2, dest)


def _sc_gather(ys, dest):
    return _make_sc_movers()[1](ys, dest)


def _ffn_body(be_ref, xs_ref, gbw_ref, gdw_ref, guw_ref, ubw_ref, udw_ref,
              uuw_ref, dw_ref, ys_ref):
    @pl.when(pl.program_id(0) < be_ref[NB])
    def _():
        ys_ref[...] = _ffn(xs_ref[...], gbw_ref[0], gdw_ref[0], guw_ref[0],
                           ubw_ref[0], udw_ref[0], uuw_ref[0], dw_ref[0])


def _grouped_ffn(be, xs, params):
    ew = lambda shape: pl.BlockSpec(
        shape, lambda b, be_ref: (be_ref[b],) + (0,) * (len(shape) - 1))
    grid_spec = pltpu.PrefetchScalarGridSpec(
        num_scalar_prefetch=1,
        grid=(NB,),
        in_specs=[
            pl.BlockSpec((BT, D), lambda b, be_ref: (b, 0)),
            ew((1, DFF, D)),
            ew((1, RANK, D)),
            ew((1, DFF, RANK)),
            ew((1, DFF, D)),
            ew((1, RANK, D)),
            ew((1, DFF, RANK)),
            ew((1, D, DFF)),
        ],
        out_specs=pl.BlockSpec((BT, D), lambda b, be_ref: (b, 0)),
    )
    return pl.pallas_call(
        _ffn_body,
        grid_spec=grid_spec,
        out_shape=jax.ShapeDtypeStruct((TPAD, D), jnp.float32),
        compiler_params=pltpu.CompilerParams(
            vmem_limit_bytes=62 * 1024 * 1024),
    )(be, xs, params['gate_base_w'], params['gate_down_w'],
      params['gate_up_w'], params['up_base_w'], params['up_down_w'],
      params['up_up_w'], params['down_w'])


@jax.jit
def _moe(x, params):
    x2 = x.reshape(T, D)
    dest_b, be_b, aux = _router_plan(x2, params['router_w'])
    dest = dest_b[:, 0]
    be = be_b[:, 0]
    xs = _sc_scatter(x2, dest)
    ys = _grouped_ffn(be, xs, params)
    out2 = _sc_gather(ys, dest)
    return out2.reshape(x.shape), aux[0, 0]


def kernel(x, params):
    return _moe(x, params)


# speedup vs baseline: 1.3203x; 1.3203x over previous
import functools

import jax
import jax.numpy as jnp
from jax import lax
from jax.experimental import pallas as pl
from jax.experimental.pallas import tpu as pltpu
from jax.experimental.pallas import tpu_sc as plsc

E = 16
D = 768
DFF = 2048
RANK = 16
SCALING = 32.0 / 16.0
GAMMA = 0.1
T = 2048
BT = 128
NB = T // BT + E - 1
TPAD = NB * BT
NW = 32
TW = T // NW


def _era(h):
    return jax.nn.gelu(h, approximate=True) + GAMMA * jax.nn.softplus(h)


def _nt(a, b):
    return jax.lax.dot_general(a, b, (((1,), (1,)), ((), ())),
                               preferred_element_type=jnp.float32)


def _ffn(h, gbw, gdw, guw, ubw, udw, uuw, dw):
    gate = _era(_nt(h, gbw) + SCALING * _nt(_nt(h, gdw), guw))
    up = _nt(h, ubw) + SCALING * _nt(_nt(h, udw), uuw)
    return _nt(gate * up, dw)


def _router_body(x_ref, rw_ref, dest_ref, be_ref, aux_ref):
    logits = _nt(x_ref[...], rw_ref[...])
    m = jnp.max(logits, axis=-1, keepdims=True)
    p = jnp.exp(logits - m)
    probs = p / jnp.sum(p, axis=-1, keepdims=True)
    pmax = jnp.max(probs, axis=-1, keepdims=True)
    lanes = lax.broadcasted_iota(jnp.int32, (T, E), 1)
    idx = jnp.min(jnp.where(probs >= pmax, lanes, E), axis=-1, keepdims=True)
    oneh = (lanes == idx).astype(jnp.float32)

    counts = jnp.sum(oneh, axis=0, keepdims=True)
    p_sum = jnp.sum(probs, axis=0, keepdims=True)
    aux_ref[0, 0] = E * jnp.sum((counts / T) * (p_sum / T))

    pc = jnp.ceil(counts * (1.0 / BT)) * BT
    ii = lax.broadcasted_iota(jnp.int32, (E, E), 0)
    jj = lax.broadcasted_iota(jnp.int32, (E, E), 1)
    upper = (ii < jj).astype(jnp.float32)
    po_excl = jnp.dot(pc, upper, preferred_element_type=jnp.float32)

    ci = lax.broadcasted_iota(jnp.int32, (BT, BT), 0)
    cj = lax.broadcasted_iota(jnp.int32, (BT, BT), 1)
    lstrict = (cj < ci).astype(jnp.float32)
    carry = jnp.zeros((1, E), jnp.float32)
    pieces = []
    for c in range(T // BT):
        oc = oneh[c * BT:(c + 1) * BT]
        cum = jnp.dot(lstrict, oc, preferred_element_type=jnp.float32) + carry
        pieces.append(jnp.sum(oc * (cum + po_excl), axis=-1, keepdims=True))
        carry = carry + jnp.sum(oc, axis=0, keepdims=True)
    dest = jnp.concatenate(pieces, axis=0)
    dest_ref[...] = jnp.broadcast_to(dest, (T, 128)).astype(jnp.int32)

    total_pad = jnp.sum(pc)
    bi = lax.broadcasted_iota(jnp.int32, (NB + 1, 1), 0)
    ge = (po_excl <= (bi * BT).astype(jnp.float32)).astype(jnp.float32)
    be = jnp.sum(ge, axis=-1, keepdims=True) - 1.0
    nb_live = total_pad * (1.0 / BT)
    be = jnp.where(bi == NB, nb_live, be)
    be_ref[...] = jnp.broadcast_to(be, (NB + 1, 128)).astype(jnp.int32)


def _router_plan(x2, rw):
    return pl.pallas_call(
        _router_body,
        in_specs=[pl.BlockSpec((T, D), lambda: (0, 0)),
                  pl.BlockSpec((E, D), lambda: (0, 0))],
        out_specs=[
            pl.BlockSpec((T, 128), lambda: (0, 0)),
            pl.BlockSpec((NB + 1, 128), lambda: (0, 0)),
            pl.BlockSpec((1, 1), lambda: (0, 0), memory_space=pltpu.SMEM),
        ],
        out_shape=[
            jax.ShapeDtypeStruct((T, 128), jnp.int32),
            jax.ShapeDtypeStruct((NB + 1, 128), jnp.int32),
            jax.ShapeDtypeStruct((1, 1), jnp.float32),
        ],
    )(x2, rw)


def _sc_scratch():
    return [pltpu.VMEM((TW,), jnp.int32),
            pltpu.VMEM((TW, D), jnp.float32),
            pltpu.SemaphoreType.DMA]


@functools.lru_cache(maxsize=None)
def _make_sc_movers():
    mesh = plsc.VectorSubcoreMesh(core_axis_name="c", subcore_axis_name="s")

    @functools.partial(pl.kernel, mesh=mesh,
                       out_type=jax.ShapeDtypeStruct((TPAD, D), jnp.float32),
                       scratch_types=_sc_scratch())
    def scatter_k(x_hbm, dest_hbm, xs_hbm, idx_v, rows_v, sem):
        wid = lax.axis_index("s") * 2 + lax.axis_index("c")
        base = wid * TW
        pltpu.sync_copy(dest_hbm.at[pl.ds(base, TW)], idx_v)
        pltpu.sync_copy(x_hbm.at[pl.ds(base, TW)], rows_v)
        pltpu.async_copy(rows_v, xs_hbm.at[idx_v], sem).wait()

    @functools.partial(pl.kernel, mesh=mesh,
                       out_type=jax.ShapeDtypeStruct((T, D), jnp.float32),
                       scratch_types=_sc_scratch())
    def gather_k(ys_hbm, dest_hbm, out_hbm, idx_v, rows_v, sem):
        wid = lax.axis_index("s") * 2 + lax.axis_index("c")
        base = wid * TW
        pltpu.sync_copy(dest_hbm.at[pl.ds(base, TW)], idx_v)
        pltpu.async_copy(ys_hbm.at[idx_v], rows_v, sem).wait()
        pltpu.sync_copy(rows_v, out_hbm.at[pl.ds(base, TW)])

    return scatter_k, gather_k


def _sc_scatter(x2, dest):
    return _make_sc_movers()[0](x2, dest)


def _sc_gather(ys, dest):
    return _make_sc_movers()[1](ys, dest)


def _ffn_body(be_ref, xs_ref, gbw_ref, gdw_ref, guw_ref, ubw_ref, udw_ref,
              uuw_ref, dw_ref, ys_ref):
    @pl.when(pl.program_id(0) < be_ref[NB])
    def _():
        ys_ref[...] = _ffn(xs_ref[...], gbw_ref[0], gdw_ref[0], guw_ref[0],
                           ubw_ref[0], udw_ref[0], uuw_ref[0], dw_ref[0])


def _grouped_ffn(be, xs, params):
    ew = lambda shape: pl.BlockSpec(
        shape, lambda b, be_ref: (be_ref[b],) + (0,) * (len(shape) - 1))
    grid_spec = pltpu.PrefetchScalarGridSpec(
        num_scalar_prefetch=1,
        grid=(NB,),
        in_specs=[
            pl.BlockSpec((BT, D),
                         lambda b, be_ref: (jnp.minimum(b, be_ref[NB] - 1), 0)),
            ew((1, DFF, D)),
            ew((1, RANK, D)),
            ew((1, DFF, RANK)),
            ew((1, DFF, D)),
            ew((1, RANK, D)),
            ew((1, DFF, RANK)),
            ew((1, D, DFF)),
        ],
        out_specs=pl.BlockSpec((BT, D), lambda b, be_ref: (b, 0)),
    )
    return pl.pallas_call(
        _ffn_body,
        grid_spec=grid_spec,
        out_shape=jax.ShapeDtypeStruct((TPAD, D), jnp.float32),
        compiler_params=pltpu.CompilerParams(
            vmem_limit_bytes=62 * 1024 * 1024),
    )(be, xs, params['gate_base_w'], params['gate_down_w'],
      params['gate_up_w'], params['up_base_w'], params['up_down_w'],
      params['up_up_w'], params['down_w'])


@jax.jit
def _moe(x, params):
    x2 = x.reshape(T, D)
    dest_b, be_b, aux = _router_plan(x2, params['router_w'])
    dest = dest_b[:, 0]
    be = be_b[:, 0]
    xs = _sc_scatter(x2, dest)
    ys = _grouped_ffn(be, xs, params)
    out2 = _sc_gather(ys, dest)
    return out2.reshape(x.shape), aux[0, 0]


def kernel(x, params):
    return _moe(x, params)
